# baseline (speedup 1.0000x reference)
import functools

import jax
import jax.numpy as jnp
from jax.experimental import pallas as pl

_EMA = 2.0 / (2000 + 1)
_T = 1.0
_C = 1.4426950408889634 / _T


def _exp2_neg_dist(a, qaug):
    a2 = jnp.sum(a * a, axis=1, keepdims=True)
    ones = jnp.ones_like(a2)
    aug = jnp.concatenate([a, a2, ones], axis=1)
    d2 = jnp.dot(aug, qaug,
                 preferred_element_type=jnp.float32)
    d2 = jnp.maximum(d2, 1e-12)
    return jnp.exp2((-_C * d2) * jax.lax.rsqrt(d2))


def _stats_body(qaug_ref, a_ref, z_ref):
    i = pl.program_id(0)
    s = _exp2_neg_dist(a_ref[...], qaug_ref[...])
    s16 = s.astype(jnp.bfloat16)
    ones_row = jnp.ones((1, s.shape[0]), jnp.bfloat16)
    part = jnp.dot(ones_row, s16,
                   preferred_element_type=jnp.float32)

    @pl.when(i == 0)
    def _init():
        z_ref[...] = part

    @pl.when(i > 0)
    def _acc():
        z_ref[...] += part


def _update_body(qaug_ref, qall_ref, a_ref, mc_ref, out_ref, *, d, nc):
    a = a_ref[...]
    s = _exp2_neg_dist(a, qaug_ref[...])
    p = jnp.dot(s.astype(jnp.bfloat16), qall_ref[...],
                preferred_element_type=jnp.float32)
    wsum = p[:, d + nc:]
    scale = 1.0 - wsum
    am = jnp.concatenate([a, mc_ref[...]], axis=1)
    out_ref[...] = am * scale + p[:, :d + nc]


@jax.jit
def kernel(query_address, query_content, A, Mc):
    b, d = query_address.shape
    m = A.shape[0]
    nc = query_content.shape[1]

    q2 = jnp.sum(query_address * query_address, axis=1)[None, :]
    qaug = jnp.concatenate(
        [-2.0 * query_address.T, jnp.ones((1, b), jnp.float32), q2], axis=0)
    qall = jnp.concatenate(
        [query_address, query_content, jnp.ones((b, 1), jnp.float32)], axis=1)

    mt = 2000 if m % 2000 == 0 else (1000 if m % 1000 == 0 else m)
    nt = m // mt

    full = lambda shape: pl.BlockSpec(shape, lambda i: (0, 0))
    z = pl.pallas_call(
        _stats_body,
        grid=(nt,),
        in_specs=[full((d + 2, b)),
                  pl.BlockSpec((mt, d), lambda i: (i, 0))],
        out_specs=full((1, b)),
        out_shape=jax.ShapeDtypeStruct((1, b), jnp.float32),
    )(qaug, A)

    qall_scaled = (qall * (_EMA / z[0])[:, None]).astype(jnp.bfloat16)

    out = pl.pallas_call(
        functools.partial(_update_body, d=d, nc=nc),
        grid=(nt,),
        in_specs=[full((d + 2, b)), full((b, d + nc + 1)),
                  pl.BlockSpec((mt, d), lambda i: (i, 0)),
                  pl.BlockSpec((mt, nc), lambda i: (i, 0))],
        out_specs=pl.BlockSpec((mt, d + nc), lambda i: (i, 0)),
        out_shape=jax.ShapeDtypeStruct((m, d + nc), jnp.float32),
    )(qaug, qall_scaled, A, Mc)
    return out


# speedup vs baseline: 1.0300x; 1.0300x over previous
import functools

import jax
import jax.numpy as jnp
from jax.experimental import pallas as pl

_EMA = 2.0 / (2000 + 1)
_T = 1.0
_C = 1.4426950408889634 / _T


def _exp2_neg_dist(a, qaug):
    a2 = jnp.sum(a * a, axis=1, keepdims=True)
    ones = jnp.ones_like(a2)
    aug = jnp.concatenate([a, a2, ones], axis=1)
    d2 = jnp.dot(aug, qaug,
                 preferred_element_type=jnp.float32)
    d2 = jnp.maximum(d2, 1e-12)
    return jnp.exp2((-_C * d2) * jax.lax.rsqrt(d2))


def _stats_body(qaug_ref, a_ref, z_ref):
    i = pl.program_id(0)
    s = _exp2_neg_dist(a_ref[...], qaug_ref[...])
    s16 = s.astype(jnp.bfloat16)
    ones_row = jnp.ones((1, s.shape[0]), jnp.bfloat16)
    part = jnp.dot(ones_row, s16,
                   preferred_element_type=jnp.float32)

    @pl.when(i == 0)
    def _init():
        z_ref[...] = part

    @pl.when(i > 0)
    def _acc():
        z_ref[...] += part


def _update_body(qaug_ref, qall_ref, a_ref, mc_ref, out_ref, *, d, nc):
    a = a_ref[...]
    s = _exp2_neg_dist(a, qaug_ref[...])
    p = jnp.dot(s.astype(jnp.bfloat16), qall_ref[...],
                preferred_element_type=jnp.float32)
    wsum = p[:, d + nc:]
    scale = 1.0 - wsum
    am = jnp.concatenate([a, mc_ref[...]], axis=1)
    out_ref[...] = am * scale + p[:, :d + nc]


@jax.jit
def kernel(query_address, query_content, A, Mc):
    b, d = query_address.shape
    m = A.shape[0]
    nc = query_content.shape[1]

    q2 = jnp.sum(query_address * query_address, axis=1)[None, :]
    qaug = jnp.concatenate(
        [-2.0 * query_address.T, jnp.ones((1, b), jnp.float32), q2], axis=0)
    qall = jnp.concatenate(
        [query_address, query_content, jnp.ones((b, 1), jnp.float32)], axis=1)

    mt = 4000 if m % 4000 == 0 else (1000 if m % 1000 == 0 else m)
    nt = m // mt

    full = lambda shape: pl.BlockSpec(shape, lambda i: (0, 0))
    z = pl.pallas_call(
        _stats_body,
        grid=(nt,),
        in_specs=[full((d + 2, b)),
                  pl.BlockSpec((mt, d), lambda i: (i, 0))],
        out_specs=full((1, b)),
        out_shape=jax.ShapeDtypeStruct((1, b), jnp.float32),
    )(qaug, A)

    qall_scaled = (qall * (_EMA / z[0])[:, None]).astype(jnp.bfloat16)

    out = pl.pallas_call(
        functools.partial(_update_body, d=d, nc=nc),
        grid=(nt,),
        in_specs=[full((d + 2, b)), full((b, d + nc + 1)),
                  pl.BlockSpec((mt, d), lambda i: (i, 0)),
                  pl.BlockSpec((mt, nc), lambda i: (i, 0))],
        out_specs=pl.BlockSpec((mt, d + nc), lambda i: (i, 0)),
        out_shape=jax.ShapeDtypeStruct((m, d + nc), jnp.float32),
    )(qaug, qall_scaled, A, Mc)
    return out


# speedup vs baseline: 1.1044x; 1.0722x over previous
import functools

import jax
import jax.numpy as jnp
from jax.experimental import pallas as pl

_EMA = 2.0 / (2000 + 1)
_T = 1.0
_C = 1.4426950408889634 / _T


def _stats_body(qaug_ref, a_ref, z_ref, s_ref):
    i = pl.program_id(0)
    a = a_ref[...]
    a2 = jnp.sum(a * a, axis=1, keepdims=True)
    aug = jnp.concatenate([a, a2, jnp.ones_like(a2)], axis=1)
    d2 = jnp.dot(aug, qaug_ref[...],
                 preferred_element_type=jnp.float32)
    d2 = jnp.maximum(d2, 1e-12)
    s = jnp.exp2((-_C * d2) * jax.lax.rsqrt(d2))
    s16 = s.astype(jnp.bfloat16)
    s_ref[...] = s16
    ones_row = jnp.ones((1, s.shape[0]), jnp.bfloat16)
    part = jnp.dot(ones_row, s16,
                   preferred_element_type=jnp.float32)

    @pl.when(i == 0)
    def _init():
        z_ref[...] = part

    @pl.when(i > 0)
    def _acc():
        z_ref[...] += part


def _update_body(qall_ref, s_ref, a_ref, mc_ref, out_ref, *, d, nc):
    p = jnp.dot(s_ref[...], qall_ref[...],
                preferred_element_type=jnp.float32)
    wsum = p[:, d + nc:]
    scale = 1.0 - wsum
    am = jnp.concatenate([a_ref[...], mc_ref[...]], axis=1)
    out_ref[...] = am * scale + p[:, :d + nc]


@jax.jit
def kernel(query_address, query_content, A, Mc):
    b, d = query_address.shape
    m = A.shape[0]
    nc = query_content.shape[1]

    q2 = jnp.sum(query_address * query_address, axis=1)[None, :]
    qaug = jnp.concatenate(
        [-2.0 * query_address.T, jnp.ones((1, b), jnp.float32), q2], axis=0)
    qall = jnp.concatenate(
        [query_address, query_content, jnp.ones((b, 1), jnp.float32)], axis=1)

    mt = 2000 if m % 2000 == 0 else (1000 if m % 1000 == 0 else m)
    nt = m // mt

    full = lambda shape: pl.BlockSpec(shape, lambda i: (0, 0))
    z, s16 = pl.pallas_call(
        _stats_body,
        grid=(nt,),
        in_specs=[full((d + 2, b)),
                  pl.BlockSpec((mt, d), lambda i: (i, 0))],
        out_specs=[full((1, b)),
                   pl.BlockSpec((mt, b), lambda i: (i, 0))],
        out_shape=[jax.ShapeDtypeStruct((1, b), jnp.float32),
                   jax.ShapeDtypeStruct((m, b), jnp.bfloat16)],
    )(qaug, A)

    qall_scaled = (qall * (_EMA / z[0])[:, None]).astype(jnp.bfloat16)

    out = pl.pallas_call(
        functools.partial(_update_body, d=d, nc=nc),
        grid=(nt,),
        in_specs=[full((b, d + nc + 1)),
                  pl.BlockSpec((mt, b), lambda i: (i, 0)),
                  pl.BlockSpec((mt, d), lambda i: (i, 0)),
                  pl.BlockSpec((mt, nc), lambda i: (i, 0))],
        out_specs=pl.BlockSpec((mt, d + nc), lambda i: (i, 0)),
        out_shape=jax.ShapeDtypeStruct((m, d + nc), jnp.float32),
    )(qall_scaled, s16, A, Mc)
    return out
